# baseline (speedup 1.0000x reference)
import functools

import jax
import jax.numpy as jnp
from jax import lax
from jax.experimental import pallas as pl
from jax.experimental.pallas import tpu as pltpu
from jax.experimental.pallas import tpu_sc as plsc

N = 10000
E = 320000
B = 100
DIM_IN = 128
D1 = 100
DG = 20
D1P = 112
DGP = 32
D2D = 200
DH = 64
MLP1 = 128
MLP2 = 32
EPS = 1e-5

NTILES = 32
CH = 50
NIT = (E // NTILES) // CH
NSLOT = 8
IA = 5
GA = 4
SD = NSLOT - IA
N_ACC = N + 8
RC = 80
NRC = N // RC
RR = -(-NRC // 16)


def _pad_ones(q, d, dp):
    qp = jnp.pad(q, ((0, 0), (0, dp - d)))
    col = lax.broadcasted_iota(jnp.int32, qp.shape, 1)
    return jnp.where(col == d, 1.0, qp)


def _proj1_body(x_ref, w_ref, o_ref):
    q = jnp.dot(x_ref[...], w_ref[...], preferred_element_type=jnp.float32)
    o_ref[...] = _pad_ones(q, D1, D1P)


def _mid_body(p_ref, w_ref, b_ref, o_ref):
    acc = p_ref[0] + p_ref[1]
    deg = jnp.maximum(acc[:, D1:D1 + 1], 1.0)
    h1 = jnp.maximum(acc[:, :D1] * (1.0 / deg) + b_ref[...], 0.0)
    q2 = jnp.dot(h1, w_ref[...], preferred_element_type=jnp.float32)
    o_ref[...] = _pad_ones(q2, DG, DGP)


def _tail_body(p_ref, ids_ref, b2_ref, d2_ref, wpg_ref, bpg_ref, wp2_ref,
               bp2_ref, watt_ref, wf1_ref, bf1_ref, wf2_ref,
               bf2_ref, wf3_ref, bf3_ref, o_ref):
    acc = p_ref[0] + p_ref[1]
    deg = jnp.maximum(acc[:, DG:DG + 1], 1.0)
    h2 = jnp.maximum(acc[:, :DG] * (1.0 / deg) + b2_ref[...], 0.0)
    h2 = _pad_ones(h2, DG, DGP)

    ids = ids_ref[...]
    gid = lax.broadcasted_iota(jnp.int32, (B, N), 0)
    member = (gid == ids).astype(jnp.float32)
    s = jnp.dot(member, h2, preferred_element_type=jnp.float32)
    cnt = jnp.maximum(s[:, DG:DG + 1], 1.0)
    hgf = s * (1.0 / cnt)

    h_g = jnp.dot(hgf[:, :DG], wpg_ref[...], preferred_element_type=jnp.float32) + bpg_ref[...]
    d2 = d2_ref[...]
    h_d = jnp.dot(d2, wp2_ref[...], preferred_element_type=jnp.float32) + bp2_ref[...]
    t = jnp.dot(h_g, watt_ref[...], preferred_element_type=jnp.float32)
    score = jnp.sum(t * h_d, axis=1, keepdims=True)
    a = 1.0 / (1.0 + jnp.exp(-score))
    g2 = a * d2
    g2a = jnp.concatenate([g2, jnp.ones((B, 1), jnp.float32)], axis=1)

    out1 = bf1_ref[...]
    for i in range(DG + 1):
        ti = jnp.dot(g2a, wf1_ref[i], preferred_element_type=jnp.float32)
        out1 = out1 + hgf[:, i:i + 1] * ti
    mu = jnp.mean(out1, axis=0, keepdims=True)
    var = jnp.mean((out1 - mu) ** 2, axis=0, keepdims=True)
    out1 = jnp.maximum((out1 - mu) / jnp.sqrt(var + EPS), 0.0)

    out2 = jnp.dot(out1, wf2_ref[...], preferred_element_type=jnp.float32) + bf2_ref[...]
    mu2 = jnp.mean(out2, axis=0, keepdims=True)
    var2 = jnp.mean((out2 - mu2) ** 2, axis=0, keepdims=True)
    out2 = jnp.maximum((out2 - mu2) / jnp.sqrt(var2 + EPS), 0.0)

    o_ref[...] = jnp.dot(out2, wf3_ref[...], preferred_element_type=jnp.float32) + bf3_ref[...]


@functools.cache
def _make_agg(width, ch=CH, nit=NIT, local_q=False):
    assert nit % NSLOT == 0
    mesh = plsc.VectorSubcoreMesh(core_axis_name="c", subcore_axis_name="s")

    scratch = [
        pltpu.VMEM_SHARED((N_ACC, width), jnp.float32),
        [pltpu.VMEM((2, ch), jnp.int32) for _ in range(NSLOT)],
        [pltpu.VMEM((ch, width), jnp.float32) for _ in range(NSLOT)],
        [pltpu.SemaphoreType.DMA for _ in range(NSLOT)],
        [pltpu.SemaphoreType.DMA for _ in range(NSLOT)],
        [pltpu.SemaphoreType.DMA for _ in range(NSLOT)],
        pltpu.SemaphoreType.DMA,
    ]
    if local_q:
        scratch.append(pltpu.VMEM_SHARED((N, width), jnp.float32))

    @functools.partial(
        pl.kernel,
        out_type=jax.ShapeDtypeStruct((2, N, width), jnp.float32),
        mesh=mesh,
        scratch_types=scratch,
        compiler_params=pltpu.CompilerParams(use_tc_tiling_on_sc=False),
    )
    def agg(q_hbm, sd_hbm, zer_hbm, out_hbm,
            acc_sh, sd, rows, ssem, gsem, scsem, psem, *maybe_q_sh):
        c = lax.axis_index("c")
        s = lax.axis_index("s")
        wid = s * 2 + c
        cbase = wid * nit
        q_src = maybe_q_sh[0] if local_q else q_hbm

        def idx_load(k, b):
            pltpu.async_copy(sd_hbm.at[:, cbase + k], sd[b], ssem[b])

        def idx_wait(b):
            pltpu.make_async_copy(sd_hbm.at[:, 0], sd[b], ssem[b]).wait()

        def gather_start(b):
            pltpu.async_copy(q_src.at[sd[b].at[0]], rows[b], gsem[b])

        def gather_wait(b):
            pltpu.make_async_copy(q_src.at[sd[b].at[0]], rows[b], gsem[b]).wait()

        def scat_start(b):
            pltpu.async_copy(rows[b], acc_sh.at[sd[b].at[1]], scsem[b], add=True)

        def scat_wait(b):
            pltpu.make_async_copy(rows[b], acc_sh.at[sd[b].at[1]], scsem[b]).wait()

        for j in range(RR):
            idx = s + 16 * j
            @pl.when(idx < NRC)
            def _():
                pltpu.async_copy(zer_hbm, acc_sh.at[pl.ds(idx * RC, RC)], psem)
                if local_q:
                    pltpu.async_copy(
                        q_hbm.at[pl.ds(idx * RC, RC)],
                        maybe_q_sh[0].at[pl.ds(idx * RC, RC)], psem)
        for k in range(IA):
            idx_load(k, k)
        if not local_q:
            for k in range(GA):
                idx_wait(k)
                gather_start(k)
        for j in range(RR):
            idx = s + 16 * j
            @pl.when(idx < NRC)
            def _():
                pltpu.make_async_copy(
                    zer_hbm, acc_sh.at[pl.ds(idx * RC, RC)], psem).wait()
                if local_q:
                    pltpu.make_async_copy(
                        q_hbm.at[pl.ds(idx * RC, RC)],
                        maybe_q_sh[0].at[pl.ds(idx * RC, RC)], psem).wait()
        plsc.subcore_barrier()
        if local_q:
            for k in range(GA):
                idx_wait(k)
                gather_start(k)

        def body(i, carry):
            for u in range(NSLOT):
                k = NSLOT * i + u
                v = (u + IA) % NSLOT
                w = (u + GA) % NSLOT
                @pl.when(k >= SD)
                def _():
                    scat_wait(v)
                @pl.when(k + IA < nit)
                def _():
                    idx_load(k + IA, v)
                @pl.when(k + GA < nit)
                def _():
                    idx_wait(w)
                    gather_start(w)
                gather_wait(u)
                scat_start(u)
            return carry

        lax.fori_loop(0, nit // NSLOT, body, 0)
        for d in range(SD):
            scat_wait((nit - SD + d) % NSLOT)
        plsc.subcore_barrier()
        for j in range(RR):
            idx = s + 16 * j
            @pl.when(idx < NRC)
            def _():
                pltpu.async_copy(
                    acc_sh.at[pl.ds(idx * RC, RC)],
                    out_hbm.at[c, pl.ds(idx * RC, RC)], psem)
        for j in range(RR):
            idx = s + 16 * j
            @pl.when(idx < NRC)
            def _():
                pltpu.make_async_copy(
                    acc_sh.at[pl.ds(idx * RC, RC)],
                    out_hbm.at[c, pl.ds(idx * RC, RC)], psem).wait()

    return agg


def kernel(x, edge_index, node_graph_ids, desc_2d, desc_3d,
           W_gc1, b_gc1, W_gc2, b_gc2, W_pg, b_pg, W_p2, b_p2, W_att,
           W_fc1, b_fc1, W_fc2, b_fc2, W_fc3, b_fc3):
    f32 = jnp.float32
    sd = edge_index.reshape(2, -1, CH)
    ids2d = node_graph_ids.reshape(1, N)
    zer1 = jnp.zeros((RC, D1P), f32)
    zer2 = jnp.zeros((RC, DGP), f32)

    q1 = pl.pallas_call(
        _proj1_body,
        out_shape=jax.ShapeDtypeStruct((N, D1P), f32),
    )(x, W_gc1)

    p1 = _make_agg(D1P)(q1, sd, zer1)

    q2 = pl.pallas_call(
        _mid_body,
        out_shape=jax.ShapeDtypeStruct((N, DGP), f32),
    )(p1, W_gc2, b_gc1.reshape(1, D1))

    p2 = _make_agg(DGP, local_q=True)(q2, sd, zer2)

    out = pl.pallas_call(
        _tail_body,
        out_shape=jax.ShapeDtypeStruct((B, 1), f32),
    )(p2, ids2d, b_gc2.reshape(1, DG), desc_2d, W_pg, b_pg.reshape(1, DH),
      W_p2, b_p2.reshape(1, DH), W_att,
      W_fc1.reshape(DG + 1, D2D + 1, MLP1),
      b_fc1.reshape(1, MLP1), W_fc2, b_fc2.reshape(1, MLP2),
      W_fc3, b_fc3.reshape(1, 1))
    return out


# speedup vs baseline: 1.1104x; 1.1104x over previous
import functools

import jax
import jax.numpy as jnp
from jax import lax
from jax.experimental import pallas as pl
from jax.experimental.pallas import tpu as pltpu
from jax.experimental.pallas import tpu_sc as plsc

N = 10000
E = 320000
B = 100
DIM_IN = 128
D1 = 100
DG = 20
D1P = 112
DGP = 32
D2D = 200
DH = 64
MLP1 = 128
MLP2 = 32
EPS = 1e-5

NTILES = 32
CH = 100
NIT = (E // NTILES) // CH
NSLOT = 5
IA = 3
GA = 2
SD = NSLOT - IA
N_ACC = N + 8
RC = 80
NRC = N // RC
RR = -(-NRC // 16)


def _pad_ones(q, d, dp):
    qp = jnp.pad(q, ((0, 0), (0, dp - d)))
    col = lax.broadcasted_iota(jnp.int32, qp.shape, 1)
    return jnp.where(col == d, 1.0, qp)


def _proj1_body(x_ref, w_ref, o_ref):
    q = jnp.dot(x_ref[...], w_ref[...], preferred_element_type=jnp.float32)
    o_ref[...] = _pad_ones(q, D1, D1P)


def _mid_body(p_ref, w_ref, b_ref, o_ref):
    acc = p_ref[0] + p_ref[1]
    deg = jnp.maximum(acc[:, D1:D1 + 1], 1.0)
    h1 = jnp.maximum(acc[:, :D1] * (1.0 / deg) + b_ref[...], 0.0)
    q2 = jnp.dot(h1, w_ref[...], preferred_element_type=jnp.float32)
    o_ref[...] = _pad_ones(q2, DG, DGP)


def _tail_body(p_ref, ids_ref, b2_ref, d2_ref, wpg_ref, bpg_ref, wp2_ref,
               bp2_ref, watt_ref, wf1_ref, bf1_ref, wf2_ref,
               bf2_ref, wf3_ref, bf3_ref, o_ref):
    acc = p_ref[0] + p_ref[1]
    deg = jnp.maximum(acc[:, DG:DG + 1], 1.0)
    h2 = jnp.maximum(acc[:, :DG] * (1.0 / deg) + b2_ref[...], 0.0)
    h2 = _pad_ones(h2, DG, DGP)

    ids = ids_ref[...]
    gid = lax.broadcasted_iota(jnp.int32, (B, N), 0)
    member = (gid == ids).astype(jnp.float32)
    s = jnp.dot(member, h2, preferred_element_type=jnp.float32)
    cnt = jnp.maximum(s[:, DG:DG + 1], 1.0)
    hgf = s * (1.0 / cnt)

    h_g = jnp.dot(hgf[:, :DG], wpg_ref[...], preferred_element_type=jnp.float32) + bpg_ref[...]
    d2 = d2_ref[...]
    h_d = jnp.dot(d2, wp2_ref[...], preferred_element_type=jnp.float32) + bp2_ref[...]
    t = jnp.dot(h_g, watt_ref[...], preferred_element_type=jnp.float32)
    score = jnp.sum(t * h_d, axis=1, keepdims=True)
    a = 1.0 / (1.0 + jnp.exp(-score))
    g2 = a * d2
    g2a = jnp.concatenate([g2, jnp.ones((B, 1), jnp.float32)], axis=1)

    out1 = bf1_ref[...]
    for i in range(DG + 1):
        ti = jnp.dot(g2a, wf1_ref[i], preferred_element_type=jnp.float32)
        out1 = out1 + hgf[:, i:i + 1] * ti
    mu = jnp.mean(out1, axis=0, keepdims=True)
    var = jnp.mean((out1 - mu) ** 2, axis=0, keepdims=True)
    out1 = jnp.maximum((out1 - mu) / jnp.sqrt(var + EPS), 0.0)

    out2 = jnp.dot(out1, wf2_ref[...], preferred_element_type=jnp.float32) + bf2_ref[...]
    mu2 = jnp.mean(out2, axis=0, keepdims=True)
    var2 = jnp.mean((out2 - mu2) ** 2, axis=0, keepdims=True)
    out2 = jnp.maximum((out2 - mu2) / jnp.sqrt(var2 + EPS), 0.0)

    o_ref[...] = jnp.dot(out2, wf3_ref[...], preferred_element_type=jnp.float32) + bf3_ref[...]


@functools.cache
def _make_agg(width, ch=CH, nit=NIT, local_q=False):
    assert nit % NSLOT == 0
    mesh = plsc.VectorSubcoreMesh(core_axis_name="c", subcore_axis_name="s")

    scratch = [
        pltpu.VMEM_SHARED((N_ACC, width), jnp.float32),
        [pltpu.VMEM((2, ch), jnp.int32) for _ in range(NSLOT)],
        [pltpu.VMEM((ch, width), jnp.float32) for _ in range(NSLOT)],
        [pltpu.SemaphoreType.DMA for _ in range(NSLOT)],
        [pltpu.SemaphoreType.DMA for _ in range(NSLOT)],
        [pltpu.SemaphoreType.DMA for _ in range(NSLOT)],
        pltpu.SemaphoreType.DMA,
    ]
    if local_q:
        scratch.append(pltpu.VMEM_SHARED((N, width), jnp.float32))

    @functools.partial(
        pl.kernel,
        out_type=jax.ShapeDtypeStruct((2, N, width), jnp.float32),
        mesh=mesh,
        scratch_types=scratch,
        compiler_params=pltpu.CompilerParams(use_tc_tiling_on_sc=False),
    )
    def agg(q_hbm, sd_hbm, zer_hbm, out_hbm,
            acc_sh, sd, rows, ssem, gsem, scsem, psem, *maybe_q_sh):
        c = lax.axis_index("c")
        s = lax.axis_index("s")
        wid = s * 2 + c
        cbase = wid * nit
        q_src = maybe_q_sh[0] if local_q else q_hbm

        def idx_load(k, b):
            pltpu.async_copy(sd_hbm.at[:, cbase + k], sd[b], ssem[b])

        def idx_wait(b):
            pltpu.make_async_copy(sd_hbm.at[:, 0], sd[b], ssem[b]).wait()

        def gather_start(b):
            pltpu.async_copy(q_src.at[sd[b].at[0]], rows[b], gsem[b])

        def gather_wait(b):
            pltpu.make_async_copy(q_src.at[sd[b].at[0]], rows[b], gsem[b]).wait()

        def scat_start(b):
            pltpu.async_copy(rows[b], acc_sh.at[sd[b].at[1]], scsem[b], add=True)

        def scat_wait(b):
            pltpu.make_async_copy(rows[b], acc_sh.at[sd[b].at[1]], scsem[b]).wait()

        for j in range(RR):
            idx = s + 16 * j
            @pl.when(idx < NRC)
            def _():
                pltpu.async_copy(zer_hbm, acc_sh.at[pl.ds(idx * RC, RC)], psem)
                if local_q:
                    pltpu.async_copy(
                        q_hbm.at[pl.ds(idx * RC, RC)],
                        maybe_q_sh[0].at[pl.ds(idx * RC, RC)], psem)
        for k in range(IA):
            idx_load(k, k)
        if not local_q:
            for k in range(GA):
                idx_wait(k)
                gather_start(k)
        for j in range(RR):
            idx = s + 16 * j
            @pl.when(idx < NRC)
            def _():
                pltpu.make_async_copy(
                    zer_hbm, acc_sh.at[pl.ds(idx * RC, RC)], psem).wait()
                if local_q:
                    pltpu.make_async_copy(
                        q_hbm.at[pl.ds(idx * RC, RC)],
                        maybe_q_sh[0].at[pl.ds(idx * RC, RC)], psem).wait()
        plsc.subcore_barrier()
        if local_q:
            for k in range(GA):
                idx_wait(k)
                gather_start(k)

        def body(i, carry):
            for u in range(NSLOT):
                k = NSLOT * i + u
                v = (u + IA) % NSLOT
                w = (u + GA) % NSLOT
                @pl.when(k >= SD)
                def _():
                    scat_wait(v)
                @pl.when(k + IA < nit)
                def _():
                    idx_load(k + IA, v)
                @pl.when(k + GA < nit)
                def _():
                    idx_wait(w)
                    gather_start(w)
                gather_wait(u)
                scat_start(u)
            return carry

        lax.fori_loop(0, nit // NSLOT, body, 0)
        for d in range(SD):
            scat_wait((nit - SD + d) % NSLOT)
        plsc.subcore_barrier()
        for j in range(RR):
            idx = s + 16 * j
            @pl.when(idx < NRC)
            def _():
                pltpu.async_copy(
                    acc_sh.at[pl.ds(idx * RC, RC)],
                    out_hbm.at[c, pl.ds(idx * RC, RC)], psem)
        for j in range(RR):
            idx = s + 16 * j
            @pl.when(idx < NRC)
            def _():
                pltpu.make_async_copy(
                    acc_sh.at[pl.ds(idx * RC, RC)],
                    out_hbm.at[c, pl.ds(idx * RC, RC)], psem).wait()

    return agg


def kernel(x, edge_index, node_graph_ids, desc_2d, desc_3d,
           W_gc1, b_gc1, W_gc2, b_gc2, W_pg, b_pg, W_p2, b_p2, W_att,
           W_fc1, b_fc1, W_fc2, b_fc2, W_fc3, b_fc3):
    f32 = jnp.float32
    sd = edge_index.reshape(2, -1, CH)
    ids2d = node_graph_ids.reshape(1, N)
    zer1 = jnp.zeros((RC, D1P), f32)
    zer2 = jnp.zeros((RC, DGP), f32)

    q1 = pl.pallas_call(
        _proj1_body,
        out_shape=jax.ShapeDtypeStruct((N, D1P), f32),
    )(x, W_gc1)

    p1 = _make_agg(D1P)(q1, sd, zer1)

    q2 = pl.pallas_call(
        _mid_body,
        out_shape=jax.ShapeDtypeStruct((N, DGP), f32),
    )(p1, W_gc2, b_gc1.reshape(1, D1))

    p2 = _make_agg(DGP, local_q=True)(q2, sd, zer2)

    out = pl.pallas_call(
        _tail_body,
        out_shape=jax.ShapeDtypeStruct((B, 1), f32),
    )(p2, ids2d, b_gc2.reshape(1, DG), desc_2d, W_pg, b_pg.reshape(1, DH),
      W_p2, b_p2.reshape(1, DH), W_att,
      W_fc1.reshape(DG + 1, D2D + 1, MLP1),
      b_fc1.reshape(1, MLP1), W_fc2, b_fc2.reshape(1, MLP2),
      W_fc3, b_fc3.reshape(1, 1))
    return out


# speedup vs baseline: 1.1239x; 1.0122x over previous
import functools

import jax
import jax.numpy as jnp
from jax import lax
from jax.experimental import pallas as pl
from jax.experimental.pallas import tpu as pltpu
from jax.experimental.pallas import tpu_sc as plsc

N = 10000
E = 320000
B = 100
DIM_IN = 128
D1 = 100
DG = 20
D1P = 112
DGP = 32
D2D = 200
DH = 64
MLP1 = 128
MLP2 = 32
EPS = 1e-5

NTILES = 32
CH = 80
NIT = (E // NTILES) // CH
NSLOT = 5
IA = 4
GA = 3
SD = NSLOT - IA
N_ACC = N + 8
RC = 80
NRC = N // RC
RR = -(-NRC // 16)


def _pad_ones(q, d, dp):
    qp = jnp.pad(q, ((0, 0), (0, dp - d)))
    col = lax.broadcasted_iota(jnp.int32, qp.shape, 1)
    return jnp.where(col == d, 1.0, qp)


def _proj1_body(x_ref, w_ref, o_ref):
    q = jnp.dot(x_ref[...], w_ref[...], preferred_element_type=jnp.float32)
    o_ref[...] = _pad_ones(q, D1, D1P)


def _mid_body(p_ref, w_ref, b_ref, o_ref):
    acc = p_ref[0] + p_ref[1]
    deg = jnp.maximum(acc[:, D1:D1 + 1], 1.0)
    h1 = jnp.maximum(acc[:, :D1] * (1.0 / deg) + b_ref[...], 0.0)
    q2 = jnp.dot(h1, w_ref[...], preferred_element_type=jnp.float32)
    o_ref[...] = _pad_ones(q2, DG, DGP)


def _tail_body(p_ref, ids_ref, b2_ref, d2_ref, wpg_ref, bpg_ref, wp2_ref,
               bp2_ref, watt_ref, wf1_ref, bf1_ref, wf2_ref,
               bf2_ref, wf3_ref, bf3_ref, o_ref):
    acc = p_ref[0] + p_ref[1]
    deg = jnp.maximum(acc[:, DG:DG + 1], 1.0)
    h2 = jnp.maximum(acc[:, :DG] * (1.0 / deg) + b2_ref[...], 0.0)
    h2 = _pad_ones(h2, DG, DGP)

    ids = ids_ref[...]
    gid = lax.broadcasted_iota(jnp.int32, (B, N), 0)
    member = (gid == ids).astype(jnp.float32)
    s = jnp.dot(member, h2, preferred_element_type=jnp.float32)
    cnt = jnp.maximum(s[:, DG:DG + 1], 1.0)
    hgf = s * (1.0 / cnt)

    h_g = jnp.dot(hgf[:, :DG], wpg_ref[...], preferred_element_type=jnp.float32) + bpg_ref[...]
    d2 = d2_ref[...]
    h_d = jnp.dot(d2, wp2_ref[...], preferred_element_type=jnp.float32) + bp2_ref[...]
    t = jnp.dot(h_g, watt_ref[...], preferred_element_type=jnp.float32)
    score = jnp.sum(t * h_d, axis=1, keepdims=True)
    a = 1.0 / (1.0 + jnp.exp(-score))
    g2 = a * d2
    g2a = jnp.concatenate([g2, jnp.ones((B, 1), jnp.float32)], axis=1)

    out1 = bf1_ref[...]
    for i in range(DG + 1):
        ti = jnp.dot(g2a, wf1_ref[i], preferred_element_type=jnp.float32)
        out1 = out1 + hgf[:, i:i + 1] * ti
    mu = jnp.mean(out1, axis=0, keepdims=True)
    var = jnp.mean((out1 - mu) ** 2, axis=0, keepdims=True)
    out1 = jnp.maximum((out1 - mu) / jnp.sqrt(var + EPS), 0.0)

    out2 = jnp.dot(out1, wf2_ref[...], preferred_element_type=jnp.float32) + bf2_ref[...]
    mu2 = jnp.mean(out2, axis=0, keepdims=True)
    var2 = jnp.mean((out2 - mu2) ** 2, axis=0, keepdims=True)
    out2 = jnp.maximum((out2 - mu2) / jnp.sqrt(var2 + EPS), 0.0)

    o_ref[...] = jnp.dot(out2, wf3_ref[...], preferred_element_type=jnp.float32) + bf3_ref[...]


@functools.cache
def _make_agg(width, ch=CH, nit=NIT, local_q=False):
    assert nit % NSLOT == 0
    mesh = plsc.VectorSubcoreMesh(core_axis_name="c", subcore_axis_name="s")

    scratch = [
        pltpu.VMEM_SHARED((N_ACC, width), jnp.float32),
        [pltpu.VMEM((2, ch), jnp.int32) for _ in range(NSLOT)],
        [pltpu.VMEM((ch, width), jnp.float32) for _ in range(NSLOT)],
        [pltpu.SemaphoreType.DMA for _ in range(NSLOT)],
        [pltpu.SemaphoreType.DMA for _ in range(NSLOT)],
        [pltpu.SemaphoreType.DMA for _ in range(NSLOT)],
        pltpu.SemaphoreType.DMA,
    ]
    if local_q:
        scratch.append(pltpu.VMEM_SHARED((N, width), jnp.float32))

    @functools.partial(
        pl.kernel,
        out_type=jax.ShapeDtypeStruct((2, N, width), jnp.float32),
        mesh=mesh,
        scratch_types=scratch,
        compiler_params=pltpu.CompilerParams(use_tc_tiling_on_sc=False),
    )
    def agg(q_hbm, sd_hbm, zer_hbm, out_hbm,
            acc_sh, sd, rows, ssem, gsem, scsem, psem, *maybe_q_sh):
        c = lax.axis_index("c")
        s = lax.axis_index("s")
        wid = s * 2 + c
        cbase = wid * nit
        q_src = maybe_q_sh[0] if local_q else q_hbm

        def idx_load(k, b):
            pltpu.async_copy(sd_hbm.at[:, cbase + k], sd[b], ssem[b])

        def idx_wait(b):
            pltpu.make_async_copy(sd_hbm.at[:, 0], sd[b], ssem[b]).wait()

        def gather_start(b):
            pltpu.async_copy(q_src.at[sd[b].at[0]], rows[b], gsem[b])

        def gather_wait(b):
            pltpu.make_async_copy(q_src.at[sd[b].at[0]], rows[b], gsem[b]).wait()

        def scat_start(b):
            pltpu.async_copy(rows[b], acc_sh.at[sd[b].at[1]], scsem[b], add=True)

        def scat_wait(b):
            pltpu.make_async_copy(rows[b], acc_sh.at[sd[b].at[1]], scsem[b]).wait()

        for j in range(RR):
            idx = s + 16 * j
            @pl.when(idx < NRC)
            def _():
                pltpu.async_copy(zer_hbm, acc_sh.at[pl.ds(idx * RC, RC)], psem)
                if local_q:
                    pltpu.async_copy(
                        q_hbm.at[pl.ds(idx * RC, RC)],
                        maybe_q_sh[0].at[pl.ds(idx * RC, RC)], psem)
        for k in range(IA):
            idx_load(k, k)
        if not local_q:
            for k in range(GA):
                idx_wait(k)
                gather_start(k)
        for j in range(RR):
            idx = s + 16 * j
            @pl.when(idx < NRC)
            def _():
                pltpu.make_async_copy(
                    zer_hbm, acc_sh.at[pl.ds(idx * RC, RC)], psem).wait()
                if local_q:
                    pltpu.make_async_copy(
                        q_hbm.at[pl.ds(idx * RC, RC)],
                        maybe_q_sh[0].at[pl.ds(idx * RC, RC)], psem).wait()
        plsc.subcore_barrier()
        if local_q:
            for k in range(GA):
                idx_wait(k)
                gather_start(k)

        def body(i, carry):
            for u in range(NSLOT):
                k = NSLOT * i + u
                v = (u + IA) % NSLOT
                w = (u + GA) % NSLOT
                @pl.when(k >= SD)
                def _():
                    scat_wait(v)
                @pl.when(k + IA < nit)
                def _():
                    idx_load(k + IA, v)
                @pl.when(k + GA < nit)
                def _():
                    idx_wait(w)
                    gather_start(w)
                gather_wait(u)
                scat_start(u)
            return carry

        lax.fori_loop(0, nit // NSLOT, body, 0)
        for d in range(SD):
            scat_wait((nit - SD + d) % NSLOT)
        plsc.subcore_barrier()
        for j in range(RR):
            idx = s + 16 * j
            @pl.when(idx < NRC)
            def _():
                pltpu.async_copy(
                    acc_sh.at[pl.ds(idx * RC, RC)],
                    out_hbm.at[c, pl.ds(idx * RC, RC)], psem)
        for j in range(RR):
            idx = s + 16 * j
            @pl.when(idx < NRC)
            def _():
                pltpu.make_async_copy(
                    acc_sh.at[pl.ds(idx * RC, RC)],
                    out_hbm.at[c, pl.ds(idx * RC, RC)], psem).wait()

    return agg


def kernel(x, edge_index, node_graph_ids, desc_2d, desc_3d,
           W_gc1, b_gc1, W_gc2, b_gc2, W_pg, b_pg, W_p2, b_p2, W_att,
           W_fc1, b_fc1, W_fc2, b_fc2, W_fc3, b_fc3):
    f32 = jnp.float32
    sd = edge_index.reshape(2, -1, CH)
    ids2d = node_graph_ids.reshape(1, N)
    zer1 = jnp.zeros((RC, D1P), f32)
    zer2 = jnp.zeros((RC, DGP), f32)

    q1 = pl.pallas_call(
        _proj1_body,
        out_shape=jax.ShapeDtypeStruct((N, D1P), f32),
    )(x, W_gc1)

    p1 = _make_agg(D1P)(q1, sd, zer1)

    q2 = pl.pallas_call(
        _mid_body,
        out_shape=jax.ShapeDtypeStruct((N, DGP), f32),
    )(p1, W_gc2, b_gc1.reshape(1, D1))

    p2 = _make_agg(DGP, local_q=True)(q2, sd, zer2)

    out = pl.pallas_call(
        _tail_body,
        out_shape=jax.ShapeDtypeStruct((B, 1), f32),
    )(p2, ids2d, b_gc2.reshape(1, DG), desc_2d, W_pg, b_pg.reshape(1, DH),
      W_p2, b_p2.reshape(1, DH), W_att,
      W_fc1.reshape(DG + 1, D2D + 1, MLP1),
      b_fc1.reshape(1, MLP1), W_fc2, b_fc2.reshape(1, MLP2),
      W_fc3, b_fc3.reshape(1, 1))
    return out


# speedup vs baseline: 1.1709x; 1.0418x over previous
import functools

import jax
import jax.numpy as jnp
from jax import lax
from jax.experimental import pallas as pl
from jax.experimental.pallas import tpu as pltpu
from jax.experimental.pallas import tpu_sc as plsc

N = 10000
E = 320000
B = 100
DIM_IN = 128
D1 = 100
DG = 20
D1P = 104
DGP = 32
D2D = 200
DH = 64
MLP1 = 128
MLP2 = 32
EPS = 1e-5

NTILES = 32
CH = 80
NIT = (E // NTILES) // CH
NSLOT = 5
IA = 3
GA = 2
SD = NSLOT - IA
N_ACC = N + 8
RC = 80
NRC = N // RC
RR = -(-NRC // 16)


def _pad_ones(q, d, dp):
    qp = jnp.pad(q, ((0, 0), (0, dp - d)))
    col = lax.broadcasted_iota(jnp.int32, qp.shape, 1)
    return jnp.where(col == d, 1.0, qp)


def _proj1_body(x_ref, w_ref, o_ref):
    q = jnp.dot(x_ref[...], w_ref[...], preferred_element_type=jnp.float32)
    o_ref[...] = _pad_ones(q, D1, D1P)


def _mid_body(p_ref, w_ref, b_ref, o_ref):
    acc = p_ref[0] + p_ref[1]
    deg = jnp.maximum(acc[:, D1:D1 + 1], 1.0)
    h1 = jnp.maximum(acc[:, :D1] * (1.0 / deg) + b_ref[...], 0.0)
    q2 = jnp.dot(h1, w_ref[...], preferred_element_type=jnp.float32)
    o_ref[...] = _pad_ones(q2, DG, DGP)


def _tail_body(p_ref, ids_ref, b2_ref, d2_ref, wpg_ref, bpg_ref, wp2_ref,
               bp2_ref, watt_ref, wf1_ref, bf1_ref, wf2_ref,
               bf2_ref, wf3_ref, bf3_ref, o_ref):
    acc = p_ref[0] + p_ref[1]
    deg = jnp.maximum(acc[:, DG:DG + 1], 1.0)
    h2 = jnp.maximum(acc[:, :DG] * (1.0 / deg) + b2_ref[...], 0.0)
    h2 = _pad_ones(h2, DG, DGP)

    ids = ids_ref[...]
    gid = lax.broadcasted_iota(jnp.int32, (B, N), 0)
    member = (gid == ids).astype(jnp.float32)
    s = jnp.dot(member, h2, preferred_element_type=jnp.float32)
    cnt = jnp.maximum(s[:, DG:DG + 1], 1.0)
    hgf = s * (1.0 / cnt)

    h_g = jnp.dot(hgf[:, :DG], wpg_ref[...], preferred_element_type=jnp.float32) + bpg_ref[...]
    d2 = d2_ref[...]
    h_d = jnp.dot(d2, wp2_ref[...], preferred_element_type=jnp.float32) + bp2_ref[...]
    t = jnp.dot(h_g, watt_ref[...], preferred_element_type=jnp.float32)
    score = jnp.sum(t * h_d, axis=1, keepdims=True)
    a = 1.0 / (1.0 + jnp.exp(-score))
    g2 = a * d2
    g2a = jnp.concatenate([g2, jnp.ones((B, 1), jnp.float32)], axis=1)

    out1 = bf1_ref[...]
    for i in range(DG + 1):
        ti = jnp.dot(g2a, wf1_ref[i], preferred_element_type=jnp.float32)
        out1 = out1 + hgf[:, i:i + 1] * ti
    mu = jnp.mean(out1, axis=0, keepdims=True)
    var = jnp.mean((out1 - mu) ** 2, axis=0, keepdims=True)
    out1 = jnp.maximum((out1 - mu) / jnp.sqrt(var + EPS), 0.0)

    out2 = jnp.dot(out1, wf2_ref[...], preferred_element_type=jnp.float32) + bf2_ref[...]
    mu2 = jnp.mean(out2, axis=0, keepdims=True)
    var2 = jnp.mean((out2 - mu2) ** 2, axis=0, keepdims=True)
    out2 = jnp.maximum((out2 - mu2) / jnp.sqrt(var2 + EPS), 0.0)

    o_ref[...] = jnp.dot(out2, wf3_ref[...], preferred_element_type=jnp.float32) + bf3_ref[...]


@functools.cache
def _make_agg(width, ch=CH, nit=NIT, local_q=False):
    assert nit % NSLOT == 0
    mesh = plsc.VectorSubcoreMesh(core_axis_name="c", subcore_axis_name="s")

    scratch = [
        pltpu.VMEM_SHARED((N_ACC, width), jnp.float32),
        [pltpu.VMEM((2, ch), jnp.int32) for _ in range(NSLOT)],
        [pltpu.VMEM((ch, width), jnp.float32) for _ in range(NSLOT)],
        [pltpu.SemaphoreType.DMA for _ in range(NSLOT)],
        [pltpu.SemaphoreType.DMA for _ in range(NSLOT)],
        [pltpu.SemaphoreType.DMA for _ in range(NSLOT)],
        pltpu.SemaphoreType.DMA,
    ]
    if local_q:
        scratch.append(pltpu.VMEM_SHARED((N, width), jnp.float32))

    @functools.partial(
        pl.kernel,
        out_type=jax.ShapeDtypeStruct((2, N, width), jnp.float32),
        mesh=mesh,
        scratch_types=scratch,
        compiler_params=pltpu.CompilerParams(use_tc_tiling_on_sc=False),
    )
    def agg(q_hbm, sd_hbm, zer_hbm, out_hbm,
            acc_sh, sd, rows, ssem, gsem, scsem, psem, *maybe_q_sh):
        c = lax.axis_index("c")
        s = lax.axis_index("s")
        wid = s * 2 + c
        cbase = wid * nit
        q_src = maybe_q_sh[0] if local_q else q_hbm

        def idx_load(k, b):
            pltpu.async_copy(sd_hbm.at[:, cbase + k], sd[b], ssem[b])

        def idx_wait(b):
            pltpu.make_async_copy(sd_hbm.at[:, 0], sd[b], ssem[b]).wait()

        def gather_start(b):
            pltpu.async_copy(q_src.at[sd[b].at[0]], rows[b], gsem[b])

        def gather_wait(b):
            pltpu.make_async_copy(q_src.at[sd[b].at[0]], rows[b], gsem[b]).wait()

        def scat_start(b):
            pltpu.async_copy(rows[b], acc_sh.at[sd[b].at[1]], scsem[b], add=True)

        def scat_wait(b):
            pltpu.make_async_copy(rows[b], acc_sh.at[sd[b].at[1]], scsem[b]).wait()

        for j in range(RR):
            idx = s + 16 * j
            @pl.when(idx < NRC)
            def _():
                pltpu.async_copy(zer_hbm, acc_sh.at[pl.ds(idx * RC, RC)], psem)
                if local_q:
                    pltpu.async_copy(
                        q_hbm.at[pl.ds(idx * RC, RC)],
                        maybe_q_sh[0].at[pl.ds(idx * RC, RC)], psem)
        for k in range(IA):
            idx_load(k, k)
        if not local_q:
            for k in range(GA):
                idx_wait(k)
                gather_start(k)
        for j in range(RR):
            idx = s + 16 * j
            @pl.when(idx < NRC)
            def _():
                pltpu.make_async_copy(
                    zer_hbm, acc_sh.at[pl.ds(idx * RC, RC)], psem).wait()
                if local_q:
                    pltpu.make_async_copy(
                        q_hbm.at[pl.ds(idx * RC, RC)],
                        maybe_q_sh[0].at[pl.ds(idx * RC, RC)], psem).wait()
        plsc.subcore_barrier()
        if local_q:
            for k in range(GA):
                idx_wait(k)
                gather_start(k)

        def body(i, carry):
            for u in range(NSLOT):
                k = NSLOT * i + u
                v = (u + IA) % NSLOT
                w = (u + GA) % NSLOT
                @pl.when(k >= SD)
                def _():
                    scat_wait(v)
                @pl.when(k + IA < nit)
                def _():
                    idx_load(k + IA, v)
                @pl.when(k + GA < nit)
                def _():
                    idx_wait(w)
                    gather_start(w)
                gather_wait(u)
                scat_start(u)
            return carry

        lax.fori_loop(0, nit // NSLOT, body, 0)
        for d in range(SD):
            scat_wait((nit - SD + d) % NSLOT)
        plsc.subcore_barrier()
        for j in range(RR):
            idx = s + 16 * j
            @pl.when(idx < NRC)
            def _():
                pltpu.async_copy(
                    acc_sh.at[pl.ds(idx * RC, RC)],
                    out_hbm.at[c, pl.ds(idx * RC, RC)], psem)
        for j in range(RR):
            idx = s + 16 * j
            @pl.when(idx < NRC)
            def _():
                pltpu.make_async_copy(
                    acc_sh.at[pl.ds(idx * RC, RC)],
                    out_hbm.at[c, pl.ds(idx * RC, RC)], psem).wait()

    return agg


def kernel(x, edge_index, node_graph_ids, desc_2d, desc_3d,
           W_gc1, b_gc1, W_gc2, b_gc2, W_pg, b_pg, W_p2, b_p2, W_att,
           W_fc1, b_fc1, W_fc2, b_fc2, W_fc3, b_fc3):
    f32 = jnp.float32
    sd = edge_index.reshape(2, -1, CH)
    ids2d = node_graph_ids.reshape(1, N)
    zer1 = jnp.zeros((RC, D1P), f32)
    zer2 = jnp.zeros((RC, DGP), f32)

    q1 = pl.pallas_call(
        _proj1_body,
        out_shape=jax.ShapeDtypeStruct((N, D1P), f32),
    )(x, W_gc1)

    p1 = _make_agg(D1P)(q1, sd, zer1)

    q2 = pl.pallas_call(
        _mid_body,
        out_shape=jax.ShapeDtypeStruct((N, DGP), f32),
    )(p1, W_gc2, b_gc1.reshape(1, D1))

    p2 = _make_agg(DGP, local_q=True)(q2, sd, zer2)

    out = pl.pallas_call(
        _tail_body,
        out_shape=jax.ShapeDtypeStruct((B, 1), f32),
    )(p2, ids2d, b_gc2.reshape(1, DG), desc_2d, W_pg, b_pg.reshape(1, DH),
      W_p2, b_p2.reshape(1, DH), W_att,
      W_fc1.reshape(DG + 1, D2D + 1, MLP1),
      b_fc1.reshape(1, MLP1), W_fc2, b_fc2.reshape(1, MLP2),
      W_fc3, b_fc3.reshape(1, 1))
    return out
